# Initial kernel scaffold; baseline (speedup 1.0000x reference)
import jax
import jax.numpy as jnp
from jax.experimental import pallas as pl


def kernel(node_features, r_indices, c_indices, w):
    raise NotImplementedError("write your pallas kernel here")


# speedup vs baseline: 1.1327x; 1.1327x over previous
import functools

import jax
import jax.numpy as jnp
from jax import lax
from jax.experimental import pallas as pl
from jax.experimental.pallas import tpu as pltpu
from jax.experimental.pallas import tpu_sc as plsc

N_NODES = 10000
D_FEAT = 128
N_EDGES = 320000

NUM_CORES = 2
NUM_SUBCORES = 16
NUM_WORKERS = NUM_CORES * NUM_SUBCORES
EDGES_PER_WORKER = N_EDGES // NUM_WORKERS
CHUNK = 80
NCHUNK = EDGES_PER_WORKER // CHUNK
GROUPS = CHUNK // 16

_mesh = plsc.VectorSubcoreMesh(core_axis_name="c", subcore_axis_name="s")


@functools.partial(
    pl.kernel,
    mesh=_mesh,
    compiler_params=pltpu.CompilerParams(needs_layout_passes=False),
    out_type=jax.ShapeDtypeStruct((N_EDGES,), jnp.float32),
    scratch_types=[
        pltpu.VMEM((NCHUNK, CHUNK), jnp.int32),
        pltpu.VMEM((NCHUNK, CHUNK), jnp.int32),
        pltpu.VMEM((CHUNK, D_FEAT), jnp.float32),
        pltpu.VMEM((CHUNK, D_FEAT), jnp.float32),
        pltpu.VMEM((D_FEAT, 16), jnp.float32),
        pltpu.VMEM((CHUNK,), jnp.float32),
        pltpu.SemaphoreType.DMA,
        pltpu.SemaphoreType.DMA,
    ],
)
def _decoder_sc(nf_hbm, r_hbm, c_hbm, wb_hbm, out_hbm,
                ri_v, ci_v, rr_v, cr_v, wb_v, out_v, sem_r, sem_c):
    wid = lax.axis_index("s") * NUM_CORES + lax.axis_index("c")
    pltpu.sync_copy(wb_hbm, wb_v)
    pltpu.sync_copy(r_hbm.at[wid], ri_v)
    pltpu.sync_copy(c_hbm.at[wid], ci_v)
    base = wid * EDGES_PER_WORKER

    def chunk_body(i, carry):
        cp_r = pltpu.async_copy(nf_hbm.at[ri_v.at[i]], rr_v, sem_r)
        cp_c = pltpu.async_copy(nf_hbm.at[ci_v.at[i]], cr_v, sem_c)
        cp_r.wait()
        cp_c.wait()

        def group_body(g, gcarry):
            evec = lax.iota(jnp.int32, 16) + g * 16

            def d_body(d, acc):
                dvec = jnp.full((16,), d, dtype=jnp.int32)
                ar = plsc.load_gather(rr_v, [evec, dvec])
                ac = plsc.load_gather(cr_v, [evec, dvec])
                return acc + jnp.abs(ar - ac) * wb_v[d]

            acc = lax.fori_loop(0, D_FEAT, d_body,
                                jnp.zeros((16,), jnp.float32), unroll=4)
            plsc.store_scatter(out_v, [evec], acc)
            return gcarry

        lax.fori_loop(0, GROUPS, group_body, 0)
        pltpu.sync_copy(out_v, out_hbm.at[pl.ds(base + i * CHUNK, CHUNK)])
        return carry

    lax.fori_loop(0, NCHUNK, chunk_body, 0)


def kernel(node_features, r_indices, c_indices, w):
    r = r_indices.astype(jnp.int32).reshape(NUM_WORKERS, NCHUNK, CHUNK)
    c = c_indices.astype(jnp.int32).reshape(NUM_WORKERS, NCHUNK, CHUNK)
    wb = jnp.broadcast_to(w.reshape(D_FEAT, 1), (D_FEAT, 16))
    return _decoder_sc(node_features, r, c, wb)


# speedup vs baseline: 4.5804x; 4.0439x over previous
import functools

import jax
import jax.numpy as jnp
from jax import lax
from jax.experimental import pallas as pl
from jax.experimental.pallas import tpu as pltpu
from jax.experimental.pallas import tpu_sc as plsc

N_NODES = 10000
D_FEAT = 128
N_EDGES = 320000

NUM_CORES = 2
NUM_SUBCORES = 16
NUM_WORKERS = NUM_CORES * NUM_SUBCORES
EDGES_PER_WORKER = N_EDGES // NUM_WORKERS
CHUNK = 80
NCHUNK = EDGES_PER_WORKER // CHUNK
GROUPS = CHUNK // 16

_mesh = plsc.VectorSubcoreMesh(core_axis_name="c", subcore_axis_name="s")


@functools.partial(
    pl.kernel,
    mesh=_mesh,
    compiler_params=pltpu.CompilerParams(needs_layout_passes=False),
    out_type=jax.ShapeDtypeStruct((N_EDGES,), jnp.float32),
    scratch_types=[
        pltpu.VMEM((NCHUNK, CHUNK), jnp.int32),
        pltpu.VMEM((NCHUNK, CHUNK), jnp.int32),
        pltpu.VMEM((CHUNK, D_FEAT), jnp.float32),
        pltpu.VMEM((CHUNK, D_FEAT), jnp.float32),
        pltpu.VMEM((D_FEAT, 16), jnp.float32),
        pltpu.VMEM((CHUNK,), jnp.float32),
        pltpu.SemaphoreType.DMA,
        pltpu.SemaphoreType.DMA,
    ],
)
def _decoder_sc(nf_hbm, r_hbm, c_hbm, wb_hbm, out_hbm,
                ri_v, ci_v, rr_v, cr_v, wb_v, out_v, sem_r, sem_c):
    wid = lax.axis_index("s") * NUM_CORES + lax.axis_index("c")
    pltpu.sync_copy(wb_hbm, wb_v)
    pltpu.sync_copy(r_hbm.at[wid], ri_v)
    pltpu.sync_copy(c_hbm.at[wid], ci_v)
    base = wid * EDGES_PER_WORKER

    def chunk_body(i, carry):
        cp_r = pltpu.async_copy(nf_hbm.at[ri_v.at[i]], rr_v, sem_r)
        cp_c = pltpu.async_copy(nf_hbm.at[ci_v.at[i]], cr_v, sem_c)
        cp_r.wait()
        cp_c.wait()

        def group_body(g, gcarry):
            lane = lax.iota(jnp.int32, 16)
            evec = lane + g * 16

            def d_body(d, acc):
                dvec = (jnp.full((16,), d, dtype=jnp.int32) + lane) & 127
                ar = plsc.load_gather(rr_v, [evec, dvec])
                ac = plsc.load_gather(cr_v, [evec, dvec])
                return acc + jnp.abs(ar - ac) * wb_v[d]

            acc = lax.fori_loop(0, D_FEAT, d_body,
                                jnp.zeros((16,), jnp.float32), unroll=4)
            plsc.store_scatter(out_v, [evec], acc)
            return gcarry

        lax.fori_loop(0, GROUPS, group_body, 0)
        pltpu.sync_copy(out_v, out_hbm.at[pl.ds(base + i * CHUNK, CHUNK)])
        return carry

    lax.fori_loop(0, NCHUNK, chunk_body, 0)


def kernel(node_features, r_indices, c_indices, w):
    r = r_indices.astype(jnp.int32).reshape(NUM_WORKERS, NCHUNK, CHUNK)
    c = c_indices.astype(jnp.int32).reshape(NUM_WORKERS, NCHUNK, CHUNK)
    rot = (jnp.arange(D_FEAT)[:, None] + jnp.arange(16)[None, :]) % D_FEAT
    wb = w.reshape(D_FEAT)[rot]
    return _decoder_sc(node_features, r, c, wb)


# speedup vs baseline: 7.8961x; 1.7239x over previous
import functools

import jax
import jax.numpy as jnp
from jax import lax
from jax.experimental import pallas as pl
from jax.experimental.pallas import tpu as pltpu
from jax.experimental.pallas import tpu_sc as plsc

N_NODES = 10000
D_FEAT = 128
N_EDGES = 320000

NUM_CORES = 2
NUM_SUBCORES = 16
NUM_WORKERS = NUM_CORES * NUM_SUBCORES
EDGES_PER_WORKER = N_EDGES // NUM_WORKERS
CHUNK = 80
NCHUNK = EDGES_PER_WORKER // CHUNK
GROUPS = CHUNK // 16

_mesh = plsc.VectorSubcoreMesh(core_axis_name="c", subcore_axis_name="s")


@functools.partial(
    pl.kernel,
    mesh=_mesh,
    compiler_params=pltpu.CompilerParams(needs_layout_passes=False),
    out_type=jax.ShapeDtypeStruct((N_EDGES,), jnp.float32),
    scratch_types=[
        pltpu.VMEM((NCHUNK, CHUNK), jnp.int32),
        pltpu.VMEM((NCHUNK, CHUNK), jnp.int32),
        pltpu.VMEM((2, CHUNK, D_FEAT), jnp.float32),
        pltpu.VMEM((2, CHUNK, D_FEAT), jnp.float32),
        pltpu.VMEM((D_FEAT, 16), jnp.float32),
        pltpu.VMEM((EDGES_PER_WORKER,), jnp.float32),
        pltpu.SemaphoreType.DMA((2,)),
        pltpu.SemaphoreType.DMA((2,)),
    ],
)
def _decoder_sc(nf_hbm, r_hbm, c_hbm, wb_hbm, out_hbm,
                ri_v, ci_v, rr_v, cr_v, wb_v, out_v, sem_r, sem_c):
    wid = lax.axis_index("s") * NUM_CORES + lax.axis_index("c")
    pltpu.sync_copy(wb_hbm, wb_v)
    pltpu.sync_copy(r_hbm.at[wid], ri_v)
    pltpu.sync_copy(c_hbm.at[wid], ci_v)
    base = wid * EDGES_PER_WORKER
    lane = lax.iota(jnp.int32, 16)

    def start_gathers(i, b):
        pltpu.async_copy(nf_hbm.at[ri_v.at[i]], rr_v.at[b], sem_r.at[b])
        pltpu.async_copy(nf_hbm.at[ci_v.at[i]], cr_v.at[b], sem_c.at[b])

    start_gathers(0, 0)

    def chunk_body(i, carry):
        b = lax.rem(i, 2)
        nb = 1 - b

        @pl.when(i + 1 < NCHUNK)
        def _():
            start_gathers(i + 1, nb)

        pltpu.make_async_copy(nf_hbm.at[ri_v.at[i]], rr_v.at[b],
                              sem_r.at[b]).wait()
        pltpu.make_async_copy(nf_hbm.at[ci_v.at[i]], cr_v.at[b],
                              sem_c.at[b]).wait()

        bvec = jnp.full((16,), b, dtype=jnp.int32)

        def group_body(g, gcarry):
            evec = lane + g * 16

            def d_body(d, acc):
                dvec = (jnp.full((16,), d, dtype=jnp.int32) + lane) & 127
                ar = plsc.load_gather(rr_v, [bvec, evec, dvec])
                ac = plsc.load_gather(cr_v, [bvec, evec, dvec])
                return acc + jnp.abs(ar - ac) * wb_v[d]

            acc = lax.fori_loop(0, D_FEAT, d_body,
                                jnp.zeros((16,), jnp.float32), unroll=4)
            plsc.store_scatter(out_v, [evec + i * CHUNK], acc)
            return gcarry

        lax.fori_loop(0, GROUPS, group_body, 0)
        return carry

    lax.fori_loop(0, NCHUNK, chunk_body, 0)
    pltpu.sync_copy(out_v, out_hbm.at[pl.ds(base, EDGES_PER_WORKER)])


def kernel(node_features, r_indices, c_indices, w):
    r = r_indices.astype(jnp.int32).reshape(NUM_WORKERS, NCHUNK, CHUNK)
    c = c_indices.astype(jnp.int32).reshape(NUM_WORKERS, NCHUNK, CHUNK)
    rot = (jnp.arange(D_FEAT)[:, None] + jnp.arange(16)[None, :]) % D_FEAT
    wb = w.reshape(D_FEAT)[rot]
    return _decoder_sc(node_features, r, c, wb)


# speedup vs baseline: 9.1648x; 1.1607x over previous
import functools

import jax
import jax.numpy as jnp
from jax import lax
from jax.experimental import pallas as pl
from jax.experimental.pallas import tpu as pltpu
from jax.experimental.pallas import tpu_sc as plsc

N_NODES = 10000
D_FEAT = 128
N_PAIR = D_FEAT // 2
N_EDGES = 320000

NUM_CORES = 2
NUM_SUBCORES = 16
NUM_WORKERS = NUM_CORES * NUM_SUBCORES
EDGES_PER_WORKER = N_EDGES // NUM_WORKERS
CHUNK = 80
NCHUNK = EDGES_PER_WORKER // CHUNK
GROUPS = CHUNK // 16

_mesh = plsc.VectorSubcoreMesh(core_axis_name="c", subcore_axis_name="s")


@functools.partial(
    pl.kernel,
    mesh=_mesh,
    compiler_params=pltpu.CompilerParams(
        needs_layout_passes=False, use_tc_tiling_on_sc=False),
    out_type=jax.ShapeDtypeStruct((N_EDGES,), jnp.float32),
    scratch_types=[
        pltpu.VMEM((NCHUNK, CHUNK), jnp.int32),
        pltpu.VMEM((NCHUNK, CHUNK), jnp.int32),
        pltpu.VMEM((2, CHUNK, N_PAIR), jnp.float32),
        pltpu.VMEM((2, CHUNK, N_PAIR), jnp.float32),
        pltpu.VMEM((N_PAIR, 32), jnp.bfloat16),
        pltpu.VMEM((EDGES_PER_WORKER,), jnp.float32),
        pltpu.SemaphoreType.DMA((2,)),
        pltpu.SemaphoreType.DMA((2,)),
    ],
)
def _decoder_sc(nf_hbm, r_hbm, c_hbm, wb_hbm, out_hbm,
                ri_v, ci_v, rr_v, cr_v, wb_v, out_v, sem_r, sem_c):
    wid = lax.axis_index("s") * NUM_CORES + lax.axis_index("c")
    pltpu.sync_copy(wb_hbm, wb_v)
    pltpu.sync_copy(r_hbm.at[wid], ri_v)
    pltpu.sync_copy(c_hbm.at[wid], ci_v)
    base = wid * EDGES_PER_WORKER
    lane = lax.iota(jnp.int32, 16)

    def start_gathers(i, b):
        pltpu.async_copy(nf_hbm.at[ri_v.at[i]], rr_v.at[b], sem_r.at[b])
        pltpu.async_copy(nf_hbm.at[ci_v.at[i]], cr_v.at[b], sem_c.at[b])

    start_gathers(0, 0)

    def chunk_body(i, carry):
        b = lax.rem(i, 2)
        nb = 1 - b

        @pl.when(i + 1 < NCHUNK)
        def _():
            start_gathers(i + 1, nb)

        pltpu.make_async_copy(nf_hbm.at[ri_v.at[i]], rr_v.at[b],
                              sem_r.at[b]).wait()
        pltpu.make_async_copy(nf_hbm.at[ci_v.at[i]], cr_v.at[b],
                              sem_c.at[b]).wait()

        bvec = jnp.full((16,), b, dtype=jnp.int32)

        def group_body(g, gcarry):
            evec = lane + g * 16

            def p_body(p, acc):
                pvec = (jnp.full((16,), p, dtype=jnp.int32) + lane) & (N_PAIR - 1)
                ar = plsc.load_gather(rr_v, [bvec, evec, pvec])
                ac = plsc.load_gather(cr_v, [bvec, evec, pvec])
                arb = plsc.bitcast(ar, jnp.bfloat16)
                acb = plsc.bitcast(ac, jnp.bfloat16)
                m = jnp.abs(arb - acb) * wb_v[p]
                lo, hi = plsc.unpack(m, format=plsc.PackFormat.INTERLEAVED)
                return acc + lo + hi

            acc = lax.fori_loop(0, N_PAIR, p_body,
                                jnp.zeros((16,), jnp.float32), unroll=8)
            plsc.store_scatter(out_v, [evec + i * CHUNK], acc)
            return gcarry

        lax.fori_loop(0, GROUPS, group_body, 0)
        return carry

    lax.fori_loop(0, NCHUNK, chunk_body, 0)
    pltpu.sync_copy(out_v, out_hbm.at[pl.ds(base, EDGES_PER_WORKER)])


def kernel(node_features, r_indices, c_indices, w):
    r = r_indices.astype(jnp.int32).reshape(NUM_WORKERS, NCHUNK, CHUNK)
    c = c_indices.astype(jnp.int32).reshape(NUM_WORKERS, NCHUNK, CHUNK)
    nf_bf = node_features.astype(jnp.bfloat16)
    nf_packed = lax.bitcast_convert_type(
        nf_bf.reshape(N_NODES, N_PAIR, 2), jnp.float32)
    wf = w.reshape(D_FEAT).astype(jnp.bfloat16)
    q = (jnp.arange(N_PAIR)[:, None] + jnp.arange(16)[None, :]) % N_PAIR
    wb = jnp.stack([wf[2 * q], wf[2 * q + 1]], axis=-1).reshape(N_PAIR, 32)
    return _decoder_sc(nf_packed, r, c, wb)


# speedup vs baseline: 9.3301x; 1.0180x over previous
import functools

import jax
import jax.numpy as jnp
from jax import lax
from jax.experimental import pallas as pl
from jax.experimental.pallas import tpu as pltpu
from jax.experimental.pallas import tpu_sc as plsc

N_NODES = 10000
D_FEAT = 128
N_PAIR = D_FEAT // 2
N_EDGES = 320000

NUM_CORES = 2
NUM_SUBCORES = 16
NUM_WORKERS = NUM_CORES * NUM_SUBCORES
EDGES_PER_WORKER = N_EDGES // NUM_WORKERS
CHUNK = 80
NCHUNK = EDGES_PER_WORKER // CHUNK
GROUPS = CHUNK // 16
ROWS_PER_STAGE = 125

_mesh = plsc.VectorSubcoreMesh(core_axis_name="c", subcore_axis_name="s")


@functools.partial(
    pl.kernel,
    mesh=_mesh,
    compiler_params=pltpu.CompilerParams(
        needs_layout_passes=False, use_tc_tiling_on_sc=False),
    out_type=jax.ShapeDtypeStruct((N_EDGES,), jnp.float32),
    scratch_types=[
        pltpu.VMEM((NCHUNK, CHUNK), jnp.int32),
        pltpu.VMEM((NCHUNK, CHUNK), jnp.int32),
        pltpu.VMEM((2, CHUNK, N_PAIR), jnp.float32),
        pltpu.VMEM((2, CHUNK, N_PAIR), jnp.float32),
        pltpu.VMEM((N_PAIR, 32), jnp.bfloat16),
        pltpu.VMEM((EDGES_PER_WORKER,), jnp.float32),
        pltpu.VMEM((ROWS_PER_STAGE, N_PAIR), jnp.float32),
        pltpu.VMEM_SHARED((N_NODES, N_PAIR), jnp.float32),
        pltpu.SemaphoreType.DMA((2,)),
        pltpu.SemaphoreType.DMA((2,)),
    ],
)
def _decoder_sc(nf_hbm, r_hbm, c_hbm, wb_hbm, out_hbm,
                ri_v, ci_v, rr_v, cr_v, wb_v, out_v, stage_v, table_sh,
                sem_r, sem_c):
    sid = lax.axis_index("s")
    wid = sid * NUM_CORES + lax.axis_index("c")
    pltpu.sync_copy(wb_hbm, wb_v)
    pltpu.sync_copy(r_hbm.at[wid], ri_v)
    pltpu.sync_copy(c_hbm.at[wid], ci_v)

    rows_per_sub = N_NODES // NUM_SUBCORES

    def stage_body(k, carry):
        off = sid * rows_per_sub + k * ROWS_PER_STAGE
        pltpu.sync_copy(nf_hbm.at[pl.ds(off, ROWS_PER_STAGE)], stage_v)
        pltpu.sync_copy(stage_v, table_sh.at[pl.ds(off, ROWS_PER_STAGE)])
        return carry

    lax.fori_loop(0, rows_per_sub // ROWS_PER_STAGE, stage_body, 0)
    plsc.subcore_barrier()

    base = wid * EDGES_PER_WORKER
    lane = lax.iota(jnp.int32, 16)

    def start_gathers(i, b):
        pltpu.async_copy(table_sh.at[ri_v.at[i]], rr_v.at[b], sem_r.at[b])
        pltpu.async_copy(table_sh.at[ci_v.at[i]], cr_v.at[b], sem_c.at[b])

    start_gathers(0, 0)

    def chunk_body(i, carry):
        b = lax.rem(i, 2)
        nb = 1 - b

        @pl.when(i + 1 < NCHUNK)
        def _():
            start_gathers(i + 1, nb)

        pltpu.make_async_copy(table_sh.at[ri_v.at[i]], rr_v.at[b],
                              sem_r.at[b]).wait()
        pltpu.make_async_copy(table_sh.at[ci_v.at[i]], cr_v.at[b],
                              sem_c.at[b]).wait()

        bvec = jnp.full((16,), b, dtype=jnp.int32)

        def group_body(g, gcarry):
            evec = lane + g * 16

            def p_body(p, acc):
                pvec = (jnp.full((16,), p, dtype=jnp.int32) + lane) & (N_PAIR - 1)
                ar = plsc.load_gather(rr_v, [bvec, evec, pvec])
                ac = plsc.load_gather(cr_v, [bvec, evec, pvec])
                arb = plsc.bitcast(ar, jnp.bfloat16)
                acb = plsc.bitcast(ac, jnp.bfloat16)
                m = jnp.abs(arb - acb) * wb_v[p]
                lo, hi = plsc.unpack(m, format=plsc.PackFormat.INTERLEAVED)
                return acc + lo + hi

            acc = lax.fori_loop(0, N_PAIR, p_body,
                                jnp.zeros((16,), jnp.float32), unroll=8)
            plsc.store_scatter(out_v, [evec + i * CHUNK], acc)
            return gcarry

        lax.fori_loop(0, GROUPS, group_body, 0)
        return carry

    lax.fori_loop(0, NCHUNK, chunk_body, 0)
    pltpu.sync_copy(out_v, out_hbm.at[pl.ds(base, EDGES_PER_WORKER)])


def kernel(node_features, r_indices, c_indices, w):
    r = r_indices.astype(jnp.int32).reshape(NUM_WORKERS, NCHUNK, CHUNK)
    c = c_indices.astype(jnp.int32).reshape(NUM_WORKERS, NCHUNK, CHUNK)
    nf_bf = node_features.astype(jnp.bfloat16)
    nf_packed = lax.bitcast_convert_type(
        nf_bf.reshape(N_NODES, N_PAIR, 2), jnp.float32)
    wf = w.reshape(D_FEAT).astype(jnp.bfloat16)
    q = (jnp.arange(N_PAIR)[:, None] + jnp.arange(16)[None, :]) % N_PAIR
    wb = jnp.stack([wf[2 * q], wf[2 * q + 1]], axis=-1).reshape(N_PAIR, 32)
    return _decoder_sc(nf_packed, r, c, wb)


# speedup vs baseline: 10.1083x; 1.0834x over previous
import functools

import jax
import jax.numpy as jnp
from jax import lax
from jax.experimental import pallas as pl
from jax.experimental.pallas import tpu as pltpu
from jax.experimental.pallas import tpu_sc as plsc

N_NODES = 10000
D_FEAT = 128
N_PAIR = D_FEAT // 2
N_EDGES = 320000

NUM_CORES = 2
NUM_SUBCORES = 16
NUM_WORKERS = NUM_CORES * NUM_SUBCORES
EDGES_PER_WORKER = N_EDGES // NUM_WORKERS
CHUNK = 80
NCHUNK = EDGES_PER_WORKER // CHUNK
GROUPS = CHUNK // 16
ROWS_PER_STAGE = 125

_mesh = plsc.VectorSubcoreMesh(core_axis_name="c", subcore_axis_name="s")


@functools.partial(
    pl.kernel,
    mesh=_mesh,
    compiler_params=pltpu.CompilerParams(
        needs_layout_passes=False, use_tc_tiling_on_sc=False),
    out_type=jax.ShapeDtypeStruct((N_EDGES,), jnp.float32),
    scratch_types=[
        pltpu.VMEM((NCHUNK, CHUNK), jnp.int32),
        pltpu.VMEM((NCHUNK, CHUNK), jnp.int32),
        pltpu.VMEM((2, CHUNK, N_PAIR), jnp.float32),
        pltpu.VMEM((2, CHUNK, N_PAIR), jnp.float32),
        pltpu.VMEM((N_PAIR, 32), jnp.bfloat16),
        pltpu.VMEM((EDGES_PER_WORKER,), jnp.float32),
        pltpu.VMEM((ROWS_PER_STAGE, N_PAIR), jnp.float32),
        pltpu.VMEM_SHARED((N_NODES, N_PAIR), jnp.float32),
        pltpu.SemaphoreType.DMA((2,)),
        pltpu.SemaphoreType.DMA((2,)),
    ],
)
def _decoder_sc(nf_hbm, r_hbm, c_hbm, wb_hbm, out_hbm,
                ri_v, ci_v, rr_v, cr_v, wb_v, out_v, stage_v, table_sh,
                sem_r, sem_c):
    sid = lax.axis_index("s")
    wid = sid * NUM_CORES + lax.axis_index("c")
    pltpu.sync_copy(wb_hbm, wb_v)
    pltpu.sync_copy(r_hbm.at[wid], ri_v)
    pltpu.sync_copy(c_hbm.at[wid], ci_v)

    rows_per_sub = N_NODES // NUM_SUBCORES

    def stage_body(k, carry):
        off = sid * rows_per_sub + k * ROWS_PER_STAGE
        pltpu.sync_copy(nf_hbm.at[pl.ds(off, ROWS_PER_STAGE)], stage_v)
        pltpu.sync_copy(stage_v, table_sh.at[pl.ds(off, ROWS_PER_STAGE)])
        return carry

    lax.fori_loop(0, rows_per_sub // ROWS_PER_STAGE, stage_body, 0)
    plsc.subcore_barrier()

    base = wid * EDGES_PER_WORKER
    lane = lax.iota(jnp.int32, 16)

    def start_gathers(i, b):
        pltpu.async_copy(table_sh.at[ri_v.at[i]], rr_v.at[b], sem_r.at[b])
        pltpu.async_copy(table_sh.at[ci_v.at[i]], cr_v.at[b], sem_c.at[b])

    start_gathers(0, 0)

    def chunk_body(i, carry):
        b = lax.rem(i, 2)
        nb = 1 - b

        @pl.when(i + 1 < NCHUNK)
        def _():
            start_gathers(i + 1, nb)

        pltpu.make_async_copy(table_sh.at[ri_v.at[i]], rr_v.at[b],
                              sem_r.at[b]).wait()
        pltpu.make_async_copy(table_sh.at[ci_v.at[i]], cr_v.at[b],
                              sem_c.at[b]).wait()

        bvec = jnp.full((16,), b, dtype=jnp.int32)

        def group_body(g, gcarry):
            evec = lane + g * 16

            def p_body(p, accs):
                acc0, acc1 = accs
                pvec = (jnp.full((16,), p, dtype=jnp.int32) + lane) & (N_PAIR - 1)
                ar = plsc.load_gather(rr_v, [bvec, evec, pvec])
                ac = plsc.load_gather(cr_v, [bvec, evec, pvec])
                arb = plsc.bitcast(ar, jnp.bfloat16)
                acb = plsc.bitcast(ac, jnp.bfloat16)
                m = jnp.abs(arb - acb) * wb_v[p]
                lo, hi = plsc.unpack(m, format=plsc.PackFormat.INTERLEAVED)
                return acc0 + lo, acc1 + hi

            zero = jnp.zeros((16,), jnp.float32)
            acc0, acc1 = lax.fori_loop(0, N_PAIR, p_body, (zero, zero),
                                       unroll=8)
            plsc.store_scatter(out_v, [evec + i * CHUNK], acc0 + acc1)
            return gcarry

        lax.fori_loop(0, GROUPS, group_body, 0)
        return carry

    lax.fori_loop(0, NCHUNK, chunk_body, 0)
    pltpu.sync_copy(out_v, out_hbm.at[pl.ds(base, EDGES_PER_WORKER)])


def kernel(node_features, r_indices, c_indices, w):
    r = r_indices.astype(jnp.int32).reshape(NUM_WORKERS, NCHUNK, CHUNK)
    c = c_indices.astype(jnp.int32).reshape(NUM_WORKERS, NCHUNK, CHUNK)
    nf_bf = node_features.astype(jnp.bfloat16)
    nf_packed = lax.bitcast_convert_type(
        nf_bf.reshape(N_NODES, N_PAIR, 2), jnp.float32)
    wf = w.reshape(D_FEAT).astype(jnp.bfloat16)
    q = (jnp.arange(N_PAIR)[:, None] + jnp.arange(16)[None, :]) % N_PAIR
    wb = jnp.stack([wf[2 * q], wf[2 * q + 1]], axis=-1).reshape(N_PAIR, 32)
    return _decoder_sc(nf_packed, r, c, wb)
